# Initial kernel scaffold; baseline (speedup 1.0000x reference)
import jax
import jax.numpy as jnp
from jax.experimental import pallas as pl


def kernel(x, W):
    raise NotImplementedError("write your pallas kernel here")


# speedup vs baseline: 1.3043x; 1.3043x over previous
import jax
import jax.numpy as jnp
from jax.experimental import pallas as pl
from jax.experimental.pallas import tpu as pltpu

D_MODEL = 2048
N_EXP = 64
TOPK = 8
N_TOK = 8192
BLK = 512


def _router_body(x_ref, w_ref, logits_ref, scores_ref, ew_ref, ei_ref, hist_ref):
    x = x_ref[...]
    w = w_ref[...]
    logits = jax.lax.dot_general(
        x, w, (((1,), (1,)), ((), ())), preferred_element_type=jnp.float32
    )
    logits_ref[...] = logits

    m = jnp.max(logits, axis=-1, keepdims=True)
    e = jnp.exp(logits - m)
    s = e / jnp.sum(e, axis=-1, keepdims=True)
    scores_ref[...] = s

    iota = jax.lax.broadcasted_iota(jnp.int32, (BLK, N_EXP), 1)
    work = s
    hist = jnp.zeros((1, N_EXP), jnp.int32)
    ew_cols = []
    ei_cols = []
    for _ in range(TOPK):
        mx = jnp.max(work, axis=-1, keepdims=True)
        idx = jnp.min(jnp.where(work == mx, iota, N_EXP), axis=-1, keepdims=True)
        ew_cols.append(mx)
        ei_cols.append(idx)
        onehot = iota == idx
        work = jnp.where(onehot, -1.0, work)
        hist = hist + jnp.sum(onehot.astype(jnp.int32), axis=0, keepdims=True)
    ew_ref[...] = jnp.concatenate(ew_cols, axis=1)
    ei_ref[...] = jnp.concatenate(ei_cols, axis=1)

    @pl.when(pl.program_id(0) == 0)
    def _():
        hist_ref[...] = jnp.zeros_like(hist_ref)

    hist_ref[...] += hist


def kernel(x, W):
    grid = (N_TOK // BLK,)
    logits, scores, ew, ei, hist = pl.pallas_call(
        _router_body,
        grid=grid,
        in_specs=[
            pl.BlockSpec((BLK, D_MODEL), lambda i: (i, 0)),
            pl.BlockSpec((N_EXP, D_MODEL), lambda i: (0, 0)),
        ],
        out_specs=[
            pl.BlockSpec((BLK, N_EXP), lambda i: (i, 0)),
            pl.BlockSpec((BLK, N_EXP), lambda i: (i, 0)),
            pl.BlockSpec((BLK, TOPK), lambda i: (i, 0)),
            pl.BlockSpec((BLK, TOPK), lambda i: (i, 0)),
            pl.BlockSpec((1, N_EXP), lambda i: (0, 0)),
        ],
        out_shape=[
            jax.ShapeDtypeStruct((N_TOK, N_EXP), jnp.float32),
            jax.ShapeDtypeStruct((N_TOK, N_EXP), jnp.float32),
            jax.ShapeDtypeStruct((N_TOK, TOPK), jnp.float32),
            jax.ShapeDtypeStruct((N_TOK, TOPK), jnp.int32),
            jax.ShapeDtypeStruct((1, N_EXP), jnp.int32),
        ],
    )(x, W)
    return logits, scores, ew, ei, hist.reshape(N_EXP)


# speedup vs baseline: 1.5218x; 1.1668x over previous
import jax
import jax.numpy as jnp
from jax.experimental import pallas as pl
from jax.experimental.pallas import tpu as pltpu

D_MODEL = 2048
N_EXP = 64
TOPK = 8
N_TOK = 8192
BLK = 512


def _router_body(x_ref, w_ref, logits_ref, scores_ref, ew_ref, ei_ref, hist_ref):
    x = x_ref[...]
    w = w_ref[...]
    logits = jax.lax.dot_general(
        x, w, (((1,), (1,)), ((), ())), preferred_element_type=jnp.float32
    )
    logits_ref[...] = logits

    m = jnp.max(logits, axis=-1, keepdims=True)
    e = jnp.exp(logits - m)
    s = e / jnp.sum(e, axis=-1, keepdims=True)
    scores_ref[...] = s

    iota = jax.lax.broadcasted_iota(jnp.int32, (BLK, N_EXP), 1)
    bits = jax.lax.bitcast_convert_type(s, jnp.int32)
    work = (bits & ~63) | (63 - iota)
    ew_cols = []
    ei_cols = []
    for _ in range(TOPK):
        mxi = jnp.max(work, axis=-1, keepdims=True)
        ew_cols.append(jax.lax.bitcast_convert_type(mxi & ~63, jnp.float32))
        ei_cols.append(63 - (mxi & 63))
        work = jnp.where(work == mxi, -1, work)
    ew_ref[...] = jnp.concatenate(ew_cols, axis=1)
    ei_ref[...] = jnp.concatenate(ei_cols, axis=1)
    hist = jnp.sum((work == -1).astype(jnp.int32), axis=0, keepdims=True)

    @pl.when(pl.program_id(0) == 0)
    def _():
        hist_ref[...] = jnp.zeros_like(hist_ref)

    hist_ref[...] += hist


def kernel(x, W):
    grid = (N_TOK // BLK,)
    logits, scores, ew, ei, hist = pl.pallas_call(
        _router_body,
        grid=grid,
        in_specs=[
            pl.BlockSpec((BLK, D_MODEL), lambda i: (i, 0)),
            pl.BlockSpec((N_EXP, D_MODEL), lambda i: (0, 0)),
        ],
        out_specs=[
            pl.BlockSpec((BLK, N_EXP), lambda i: (i, 0)),
            pl.BlockSpec((BLK, N_EXP), lambda i: (i, 0)),
            pl.BlockSpec((BLK, TOPK), lambda i: (i, 0)),
            pl.BlockSpec((BLK, TOPK), lambda i: (i, 0)),
            pl.BlockSpec((1, N_EXP), lambda i: (0, 0)),
        ],
        out_shape=[
            jax.ShapeDtypeStruct((N_TOK, N_EXP), jnp.float32),
            jax.ShapeDtypeStruct((N_TOK, N_EXP), jnp.float32),
            jax.ShapeDtypeStruct((N_TOK, TOPK), jnp.float32),
            jax.ShapeDtypeStruct((N_TOK, TOPK), jnp.int32),
            jax.ShapeDtypeStruct((1, N_EXP), jnp.int32),
        ],
    )(x, W)
    return logits, scores, ew, ei, hist.reshape(N_EXP)


# speedup vs baseline: 1.6873x; 1.1087x over previous
import jax
import jax.numpy as jnp
from jax.experimental import pallas as pl
from jax.experimental.pallas import tpu as pltpu

D_MODEL = 2048
N_EXP = 64
TOPK = 8
N_TOK = 8192
BLK = 512


def _router_body(x_ref, w_ref, logits_ref, scores_ref, ew_ref, ei_ref, hist_ref):
    x = x_ref[...]
    w = w_ref[...]
    logits = jax.lax.dot_general(
        x, w, (((1,), (1,)), ((), ())), preferred_element_type=jnp.float32
    )
    logits_ref[...] = logits

    m = jnp.max(logits, axis=-1, keepdims=True)
    e = jnp.exp(logits - m)
    s = e / jnp.sum(e, axis=-1, keepdims=True)
    scores_ref[...] = s

    iota = jax.lax.broadcasted_iota(jnp.int32, (BLK, N_EXP), 1)
    bits = jax.lax.bitcast_convert_type(s, jnp.int32)
    work = jax.lax.bitcast_convert_type((bits & ~63) | (63 - iota), jnp.float32)
    ew_cols = []
    ei_cols = []
    for _ in range(TOPK):
        mx = jnp.max(work, axis=-1, keepdims=True)
        mxi = jax.lax.bitcast_convert_type(mx, jnp.int32)
        ew_cols.append(jax.lax.bitcast_convert_type(mxi & ~63, jnp.float32))
        ei_cols.append(63 - (mxi & 63))
        work = jnp.where(work == mx, -1.0, work)
    ew_ref[...] = jnp.concatenate(ew_cols, axis=1)
    ei_ref[...] = jnp.concatenate(ei_cols, axis=1)
    hist = jnp.sum((work == -1.0).astype(jnp.int32), axis=0, keepdims=True)

    @pl.when(pl.program_id(0) == 0)
    def _():
        hist_ref[...] = jnp.zeros_like(hist_ref)

    hist_ref[...] += hist


def kernel(x, W):
    grid = (N_TOK // BLK,)
    logits, scores, ew, ei, hist = pl.pallas_call(
        _router_body,
        grid=grid,
        in_specs=[
            pl.BlockSpec((BLK, D_MODEL), lambda i: (i, 0)),
            pl.BlockSpec((N_EXP, D_MODEL), lambda i: (0, 0)),
        ],
        out_specs=[
            pl.BlockSpec((BLK, N_EXP), lambda i: (i, 0)),
            pl.BlockSpec((BLK, N_EXP), lambda i: (i, 0)),
            pl.BlockSpec((BLK, TOPK), lambda i: (i, 0)),
            pl.BlockSpec((BLK, TOPK), lambda i: (i, 0)),
            pl.BlockSpec((1, N_EXP), lambda i: (0, 0)),
        ],
        out_shape=[
            jax.ShapeDtypeStruct((N_TOK, N_EXP), jnp.float32),
            jax.ShapeDtypeStruct((N_TOK, N_EXP), jnp.float32),
            jax.ShapeDtypeStruct((N_TOK, TOPK), jnp.float32),
            jax.ShapeDtypeStruct((N_TOK, TOPK), jnp.int32),
            jax.ShapeDtypeStruct((1, N_EXP), jnp.int32),
        ],
    )(x, W)
    return logits, scores, ew, ei, hist.reshape(N_EXP)
